# baseline (device time: 174046 ns/iter reference)
import jax
import jax.numpy as jnp
from jax import lax
from jax.experimental import pallas as pl
from jax.experimental.pallas import tpu as pltpu

B, SQ, H, D = 4, 32, 8, 128
SKV_LOCAL = 4096
HD = H * D
HQ = H * SQ
SCALE = D ** -0.5
N_CHUNKS = 4
CH = SKV_LOCAL // N_CHUNKS


def kernel(Q, K, V):
    Km = K.reshape(B, SKV_LOCAL, HD)
    Vm = V.reshape(B, SKV_LOCAL, HD)

    def body(q_ref, k_hbm, v_hbm, out_ref,
             kbuf, vbuf, qbd, o_ysend, o_yrecv, ml_ysend, ml_yrecv, obuf,
             load_sems, ysend_sems, yrecv_sems, gsend_sems, grecv_sems):
        mx = lax.axis_index("x")
        my = lax.axis_index("y")
        mz = lax.axis_index("z")
        r = mx * 2 + mz

        def k_copy(c, slot):
            return pltpu.make_async_copy(
                k_hbm.at[r, pl.ds(c * CH, CH), :], kbuf.at[slot],
                load_sems.at[slot, 0])

        def v_copy(c, slot):
            return pltpu.make_async_copy(
                v_hbm.at[r, pl.ds(c * CH, CH), :], vbuf.at[slot],
                load_sems.at[slot, 1])

        k_copy(0, 0).start()
        v_copy(0, 0).start()

        partners = [
            (mx, 1 - my, mz),
            (mx, my, 1 - mz),
            (1 - mx, my, mz),
        ]
        barrier = pltpu.get_barrier_semaphore()
        for pt in partners:
            pl.semaphore_signal(barrier, inc=1, device_id=pt,
                                device_id_type=pl.DeviceIdType.MESH)
        pl.semaphore_wait(barrier, 3)

        qbd[...] = jnp.zeros((HD, HQ), jnp.bfloat16)
        for h in range(H):
            qh = q_ref[r, :, h, :].astype(jnp.bfloat16)
            qbd[h * D:(h + 1) * D, h * SQ:(h + 1) * SQ] = qh.T

        m_run = jnp.full((1, HQ), -1e30, jnp.float32)
        l_run = jnp.zeros((1, HQ), jnp.float32)
        o_run = jnp.zeros((HQ, HD), jnp.float32)
        for c in range(N_CHUNKS):
            slot = c % 2
            k_copy(c, slot).wait()
            v_copy(c, slot).wait()
            if c + 1 < N_CHUNKS:
                k_copy(c + 1, (c + 1) % 2).start()
                v_copy(c + 1, (c + 1) % 2).start()
            kc = kbuf[slot].astype(jnp.bfloat16)
            s_t = lax.dot_general(
                kc, qbd[...], (((1,), (0,)), ((), ())),
                preferred_element_type=jnp.float32) * SCALE
            mc = jnp.max(s_t, axis=0, keepdims=True)
            m_new = jnp.maximum(m_run, mc)
            p_t = jnp.exp(s_t - m_new)
            lc = jnp.sum(p_t, axis=0, keepdims=True)
            alpha = jnp.exp(m_run - m_new)
            l_run = l_run * alpha + lc
            o_c = lax.dot_general(
                p_t.astype(jnp.bfloat16), vbuf[slot].astype(jnp.bfloat16),
                (((0,), (0,)), ((), ())),
                preferred_element_type=jnp.float32)
            o_run = o_run * alpha.T + o_c
            m_run = m_new

        o_parts = [o_run[h * SQ:(h + 1) * SQ, h * D:(h + 1) * D]
                   for h in range(H)]
        o_loc = jnp.concatenate(o_parts, axis=0)

        o_ysend[...] = o_loc.astype(jnp.bfloat16)
        ml_ysend[:, 0:1] = m_run.T
        ml_ysend[:, 1:2] = l_run.T
        rdma_o = pltpu.make_async_remote_copy(
            src_ref=o_ysend, dst_ref=o_yrecv,
            send_sem=ysend_sems.at[0], recv_sem=yrecv_sems.at[0],
            device_id=partners[0], device_id_type=pl.DeviceIdType.MESH)
        rdma_ml = pltpu.make_async_remote_copy(
            src_ref=ml_ysend, dst_ref=ml_yrecv,
            send_sem=ysend_sems.at[1], recv_sem=yrecv_sems.at[1],
            device_id=partners[0], device_id_type=pl.DeviceIdType.MESH)
        rdma_o.start()
        rdma_ml.start()
        rdma_o.wait()
        rdma_ml.wait()

        m_a, l_a = m_run.T, l_run.T
        m_b = ml_yrecv[:, 0:1]
        l_b = ml_yrecv[:, 1:2]
        m_n = jnp.maximum(m_a, m_b)
        ea = jnp.exp(m_a - m_n)
        eb = jnp.exp(m_b - m_n)
        o_comb = ea * o_loc + eb * o_yrecv[...].astype(jnp.float32)
        l_comb = ea * l_a + eb * l_b
        obuf[r] = (o_comb / l_comb).astype(jnp.bfloat16)

        rdma_z = pltpu.make_async_remote_copy(
            src_ref=obuf.at[r], dst_ref=obuf.at[r],
            send_sem=gsend_sems.at[0], recv_sem=grecv_sems.at[0],
            device_id=partners[1], device_id_type=pl.DeviceIdType.MESH)
        rdma_z.start()
        rdma_z.wait()
        rdma_x = pltpu.make_async_remote_copy(
            src_ref=obuf.at[pl.ds(mx * 2, 2)], dst_ref=obuf.at[pl.ds(mx * 2, 2)],
            send_sem=gsend_sems.at[1], recv_sem=grecv_sems.at[1],
            device_id=partners[2], device_id_type=pl.DeviceIdType.MESH)
        rdma_x.start()
        rdma_x.wait()

        out_ref[...] = obuf[...].astype(jnp.float32)

    out = pl.pallas_call(
        body,
        out_shape=jax.ShapeDtypeStruct((B, HQ, D), jnp.float32),
        in_specs=[
            pl.BlockSpec(memory_space=pltpu.VMEM),
            pl.BlockSpec(memory_space=pl.ANY),
            pl.BlockSpec(memory_space=pl.ANY),
        ],
        out_specs=pl.BlockSpec(memory_space=pltpu.VMEM),
        scratch_shapes=[
            pltpu.VMEM((2, CH, HD), jnp.float32),
            pltpu.VMEM((2, CH, HD), jnp.float32),
            pltpu.VMEM((HD, HQ), jnp.bfloat16),
            pltpu.VMEM((HQ, D), jnp.bfloat16),
            pltpu.VMEM((HQ, D), jnp.bfloat16),
            pltpu.VMEM((HQ, 2), jnp.float32),
            pltpu.VMEM((HQ, 2), jnp.float32),
            pltpu.VMEM((B, HQ, D), jnp.bfloat16),
            pltpu.SemaphoreType.DMA((2, 2)),
            pltpu.SemaphoreType.DMA((2,)),
            pltpu.SemaphoreType.DMA((2,)),
            pltpu.SemaphoreType.DMA((2,)),
            pltpu.SemaphoreType.DMA((2,)),
        ],
        compiler_params=pltpu.CompilerParams(
            collective_id=0, vmem_limit_bytes=56 * 1024 * 1024),
    )(Q, Km, Vm)
    return out.reshape(B, H, SQ, D).swapaxes(1, 2)


# device time: 43083 ns/iter; 4.0398x vs baseline; 4.0398x over previous
import jax
import jax.numpy as jnp
from jax import lax
from jax.experimental import pallas as pl
from jax.experimental.pallas import tpu as pltpu

B, SQ, H, D = 4, 32, 8, 128
SKV_LOCAL = 4096
HQ = H * SQ
SCALE = D ** -0.5
N_CHUNKS = 8
CH = SKV_LOCAL // N_CHUNKS
CHR = CH * H


def kernel(Q, K, V):
    Km = K.reshape(B, SKV_LOCAL * H, D)
    Vm = V.reshape(B, SKV_LOCAL * H, D)

    def body(q_ref, k_hbm, v_hbm, out_ref,
             kbuf, vbuf, qcat, o_ysend, o_yrecv, ml_ysend, ml_yrecv, obuf,
             load_sems, ysend_sems, yrecv_sems, gsend_sems, grecv_sems):
        mx = lax.axis_index("x")
        my = lax.axis_index("y")
        mz = lax.axis_index("z")
        r = mx * 2 + mz

        def k_copy(c, slot):
            return pltpu.make_async_copy(
                k_hbm.at[r, pl.ds(c * CHR, CHR), :], kbuf.at[slot],
                load_sems.at[slot, 0])

        def v_copy(c, slot):
            return pltpu.make_async_copy(
                v_hbm.at[r, pl.ds(c * CHR, CHR), :], vbuf.at[slot],
                load_sems.at[slot, 1])

        k_copy(0, 0).start()
        v_copy(0, 0).start()
        k_copy(1, 1).start()
        v_copy(1, 1).start()

        partners = [
            (mx, 1 - my, mz),
            (mx, my, 1 - mz),
            (1 - mx, my, mz),
            (1 - mx, my, 1 - mz),
        ]
        barrier = pltpu.get_barrier_semaphore()
        for pt in partners:
            pl.semaphore_signal(barrier, inc=1, device_id=pt,
                                device_id_type=pl.DeviceIdType.MESH)
        pl.semaphore_wait(barrier, 4)

        for h in range(H):
            qh = (q_ref[r, :, h, :] * SCALE).astype(jnp.bfloat16)
            qcat[:, h * SQ:(h + 1) * SQ] = qh.T

        ind_r = lax.broadcasted_iota(jnp.int32, (CHR, H), 0) % H
        ind_c = lax.broadcasted_iota(jnp.int32, (CHR, H), 1)
        ind = jnp.where(ind_r == ind_c, 1.0, 0.0).astype(jnp.bfloat16)
        mb_r = lax.broadcasted_iota(jnp.int32, (H, HQ), 0)
        mb_c = lax.broadcasted_iota(jnp.int32, (H, HQ), 1) // SQ
        mbias = jnp.where(mb_r == mb_c, 0.0, -1e30).astype(jnp.bfloat16)

        m_run = jnp.full((1, HQ), -1e30, jnp.float32)
        l_run = jnp.zeros((1, HQ), jnp.float32)
        o_run = jnp.zeros((HQ, D), jnp.float32)
        for c in range(N_CHUNKS):
            slot = c % 3
            k_copy(c, slot).wait()
            v_copy(c, slot).wait()
            if c + 2 < N_CHUNKS:
                k_copy(c + 2, (c + 2) % 3).start()
                v_copy(c + 2, (c + 2) % 3).start()
            kc = kbuf[slot].astype(jnp.bfloat16)
            s_all = lax.dot_general(
                kc, qcat[...], (((1,), (0,)), ((), ())),
                preferred_element_type=jnp.float32)
            s_all = s_all + lax.dot_general(
                ind, mbias, (((1,), (0,)), ((), ())),
                preferred_element_type=jnp.float32)
            mc = jnp.max(s_all, axis=0, keepdims=True)
            m_new = jnp.maximum(m_run, mc)
            p_all = jnp.exp(s_all - m_new)
            lc = jnp.sum(p_all, axis=0, keepdims=True)
            alpha = jnp.exp(m_run - m_new)
            l_run = l_run * alpha + lc
            o_c = lax.dot_general(
                p_all.astype(jnp.bfloat16), vbuf[slot].astype(jnp.bfloat16),
                (((0,), (0,)), ((), ())),
                preferred_element_type=jnp.float32)
            o_run = o_run * alpha.T + o_c
            m_run = m_new

        o_ysend[...] = o_run.astype(jnp.bfloat16)
        ml_ysend[:, 0:1] = m_run.T
        ml_ysend[:, 1:2] = l_run.T
        rdma_o = pltpu.make_async_remote_copy(
            src_ref=o_ysend, dst_ref=o_yrecv,
            send_sem=ysend_sems.at[0], recv_sem=yrecv_sems.at[0],
            device_id=partners[0], device_id_type=pl.DeviceIdType.MESH)
        rdma_ml = pltpu.make_async_remote_copy(
            src_ref=ml_ysend, dst_ref=ml_yrecv,
            send_sem=ysend_sems.at[1], recv_sem=yrecv_sems.at[1],
            device_id=partners[0], device_id_type=pl.DeviceIdType.MESH)
        rdma_o.start()
        rdma_ml.start()
        rdma_o.wait()
        rdma_ml.wait()

        m_a, l_a = m_run.T, l_run.T
        m_b = ml_yrecv[:, 0:1]
        l_b = ml_yrecv[:, 1:2]
        m_n = jnp.maximum(m_a, m_b)
        ea = jnp.exp(m_a - m_n)
        eb = jnp.exp(m_b - m_n)
        o_comb = ea * o_run + eb * o_yrecv[...].astype(jnp.float32)
        l_comb = ea * l_a + eb * l_b
        obuf[r] = (o_comb / l_comb).astype(jnp.bfloat16)

        gathers = []
        for i, pt in enumerate(partners[1:]):
            g = pltpu.make_async_remote_copy(
                src_ref=obuf.at[r], dst_ref=obuf.at[r],
                send_sem=gsend_sems.at[i], recv_sem=grecv_sems.at[i],
                device_id=pt, device_id_type=pl.DeviceIdType.MESH)
            g.start()
            gathers.append(g)
        for g in gathers:
            g.wait()

        out_ref[...] = obuf[...].astype(jnp.float32)

    out = pl.pallas_call(
        body,
        out_shape=jax.ShapeDtypeStruct((B, HQ, D), jnp.float32),
        in_specs=[
            pl.BlockSpec(memory_space=pltpu.VMEM),
            pl.BlockSpec(memory_space=pl.ANY),
            pl.BlockSpec(memory_space=pl.ANY),
        ],
        out_specs=pl.BlockSpec(memory_space=pltpu.VMEM),
        scratch_shapes=[
            pltpu.VMEM((3, CHR, D), jnp.float32),
            pltpu.VMEM((3, CHR, D), jnp.float32),
            pltpu.VMEM((D, HQ), jnp.bfloat16),
            pltpu.VMEM((HQ, D), jnp.bfloat16),
            pltpu.VMEM((HQ, D), jnp.bfloat16),
            pltpu.VMEM((HQ, 2), jnp.float32),
            pltpu.VMEM((HQ, 2), jnp.float32),
            pltpu.VMEM((B, HQ, D), jnp.bfloat16),
            pltpu.SemaphoreType.DMA((3, 2)),
            pltpu.SemaphoreType.DMA((2,)),
            pltpu.SemaphoreType.DMA((2,)),
            pltpu.SemaphoreType.DMA((3,)),
            pltpu.SemaphoreType.DMA((3,)),
        ],
        compiler_params=pltpu.CompilerParams(
            collective_id=0, vmem_limit_bytes=56 * 1024 * 1024),
    )(Q, Km, Vm)
    return out.reshape(B, H, SQ, D).swapaxes(1, 2)
